# baseline (speedup 1.0000x reference)
import jax
import jax.numpy as jnp
from jax import lax
from jax.experimental import pallas as pl
from jax.experimental.pallas import tpu as pltpu
from jax.experimental.pallas import tpu_sc as plsc

E_ENT = 100000
B = 128
CBLK = 25000
NC = 2
NS = 16
NW = NC * NS
RPW = B // NW


def _fill_body(out_ref):
    out_ref[...] = jnp.zeros((CBLK, B), jnp.float32)


def _sc_patch_body(x_hbm, o_ref, x_v, patch_v, sem):
    wid = lax.axis_index("c") * NS + lax.axis_index("s")
    pltpu.sync_copy(x_hbm, x_v.at[pl.ds(0, B)])
    lane = lax.broadcasted_iota(jnp.int32, (16,), 0)
    cps = []
    for j in range(RPW):
        i = wid * RPW + j
        xi = x_v[pl.ds(i, 16)][0]
        patch_v[j] = (lane == i % 16).astype(jnp.float32)
        cp = pltpu.make_async_copy(
            patch_v.at[j],
            o_ref.at[xi, pl.ds((i // 16) * 16, 16)],
            sem)
        cp.start()
        cps.append(cp)
    for cp in cps:
        cp.wait()


def kernel(input_x, input_r, e2triple, triple2e, r2triple, emb_table,
           W_ih, W_hh, b_ih, b_hh, W_lin, b_lin):
    x_i32 = input_x.astype(jnp.int32)
    zT = pl.pallas_call(
        _fill_body,
        grid=(E_ENT // CBLK,),
        out_specs=pl.BlockSpec((CBLK, B), lambda j: (j, 0)),
        out_shape=jax.ShapeDtypeStruct((E_ENT, B), jnp.float32),
    )()
    ref = jax.new_ref(zT)
    patch = pl.kernel(
        _sc_patch_body,
        out_type=(),
        mesh=plsc.VectorSubcoreMesh(core_axis_name="c", subcore_axis_name="s"),
        scratch_types=[
            pltpu.VMEM((B + 16,), jnp.int32),
            pltpu.VMEM((RPW, 16), jnp.float32),
            pltpu.SemaphoreType.DMA,
        ],
        compiler_params=pltpu.CompilerParams(needs_layout_passes=False),
    )
    patch(x_i32, ref)
    return ref[...].T
